# baseline (speedup 1.0000x reference)
import functools

import jax
import jax.numpy as jnp
from jax import lax
from jax.experimental import pallas as pl
from jax.experimental.pallas import tpu as pltpu
from jax.experimental.pallas import tpu_sc as plsc

_N = 10000
_F = 128
_H = 32
_B = 16
_E = 320000

_NC = 2
_NS = 16
_CH = 128
_NBUF = 4
_KTOT = -(-_E // (_NS * _CH * _NBUF)) * _NBUF
_EPAD = _NS * _KTOT * _CH
_K0 = 160
_K1 = _KTOT - _K0
_KMAX = max(_K0, _K1)
_NPAD = -(-(_N + 1) // (_NS * 8)) * _NS * 8
_RPW = _NPAD // _NS


def _segment_sum_sc(y, src3, dst3, zpad):
    width = y.shape[1]
    mesh = plsc.VectorSubcoreMesh(core_axis_name="c", subcore_axis_name="s")

    @functools.partial(
        pl.kernel,
        out_type=jax.ShapeDtypeStruct((_NC, _NPAD, width), jnp.float32),
        mesh=mesh,
        scratch_types=[
            pltpu.VMEM((_KMAX, _CH), jnp.int32),
            pltpu.VMEM((_KMAX, _CH), jnp.int32),
            pltpu.VMEM((_NBUF, _CH, width), jnp.float32),
            pltpu.VMEM_SHARED((_NPAD, width), jnp.float32),
        ] + [pltpu.SemaphoreType.DMA] * (2 * _NBUF),
        compiler_params=pltpu.CompilerParams(use_tc_tiling_on_sc=False),
    )
    def seg_kernel(y_hbm, src_hbm, dst_hbm, z_hbm, out_hbm,
                   src_v, dst_v, rows_v, acc_sh, *sems):
        gsem = sems[:_NBUF]
        ssem = sems[_NBUF:]
        cid = lax.axis_index("c")
        sid = lax.axis_index("s")
        r0 = sid * _RPW
        pltpu.sync_copy(z_hbm.at[pl.ds(r0, _RPW)], acc_sh.at[pl.ds(r0, _RPW)])
        pltpu.sync_copy(src_hbm.at[cid, sid], src_v)
        pltpu.sync_copy(dst_hbm.at[cid, sid], dst_v)
        plsc.subcore_barrier()

        n_iters = jnp.where(cid == 0, _K0 // _NBUF, _K1 // _NBUF)

        def body(i, carry):
            descs = []
            for b in range(_NBUF):
                j = i * _NBUF + b

                @pl.when(i >= 1)
                def _drain():
                    pltpu.make_async_copy(
                        rows_v.at[b], acc_sh.at[dst_v.at[0]], ssem[b]).wait()

                descs.append(pltpu.async_copy(
                    y_hbm.at[src_v.at[j]], rows_v.at[b], gsem[b]))
            for b in range(_NBUF):
                j = i * _NBUF + b
                descs[b].wait()
                pltpu.async_copy(rows_v.at[b], acc_sh.at[dst_v.at[j]],
                                 ssem[b], add=True)
            return carry

        lax.fori_loop(0, n_iters, body, 0)

        @pl.when(n_iters >= 1)
        def _final_drain():
            for b in range(_NBUF):
                pltpu.make_async_copy(
                    rows_v.at[b], acc_sh.at[dst_v.at[0]], ssem[b]).wait()

        plsc.subcore_barrier()
        pltpu.sync_copy(acc_sh.at[pl.ds(r0, _RPW)],
                        out_hbm.at[cid, pl.ds(r0, _RPW)])

    return seg_kernel(y, src3, dst3, zpad)


def _gin_layer(h, agg_ref, Wa, ba, g, be, Wb, bb):
    h = h + agg_ref[0, :_N, :] + agg_ref[1, :_N, :]
    h = jnp.dot(h, Wa, preferred_element_type=jnp.float32) + ba
    mu = jnp.mean(h, axis=0)
    var = jnp.mean((h - mu) ** 2, axis=0)
    h = (h - mu) / jnp.sqrt(var + 1e-5) * g + be
    h = jnp.maximum(h, 0.0)
    return jnp.dot(h, Wb, preferred_element_type=jnp.float32) + bb


def _mid_tc(aggL, aggR, x, W1a, b1a, g1, be1, W1b, b1b):
    def body(aggL_ref, aggR_ref, x_ref, w1a_ref, b1a_ref, g1_ref, be1_ref,
             w1b_ref, b1b_ref, h1_ref):
        agg = jnp.concatenate(
            [aggL_ref[0, :_N, :] + aggL_ref[1, :_N, :],
             aggR_ref[0, :_N, :] + aggR_ref[1, :_N, :]], axis=1)
        h = x_ref[...] + agg
        h = jnp.dot(h, w1a_ref[...], preferred_element_type=jnp.float32) + b1a_ref[...]
        mu = jnp.mean(h, axis=0)
        var = jnp.mean((h - mu) ** 2, axis=0)
        h = (h - mu) / jnp.sqrt(var + 1e-5) * g1_ref[...] + be1_ref[...]
        h = jnp.maximum(h, 0.0)
        h = jnp.dot(h, w1b_ref[...], preferred_element_type=jnp.float32) + b1b_ref[...]
        h1_ref[...] = jnp.maximum(h, 0.0)

    return pl.pallas_call(
        body, out_shape=jax.ShapeDtypeStruct((_N, _H), jnp.float32))(
            aggL, aggR, x, W1a, b1a, g1, be1, W1b, b1b)


def _head_tc(agg, h1, batch2d, W2a, b2a, g2, be2, W2b, b2b, Wc1, bc1, Wc2, bc2):
    def body(agg_ref, h1_ref, bat_ref, w2a_ref, b2a_ref, g2_ref, be2_ref,
             w2b_ref, b2b_ref, wc1_ref, bc1_ref, wc2_ref, bc2_ref, out_ref):
        h = _gin_layer(h1_ref[...], agg_ref, w2a_ref[...], b2a_ref[...],
                       g2_ref[...], be2_ref[...], w2b_ref[...], b2b_ref[...])
        bat = bat_ref[...]
        means = []
        maxs = []
        neg = jnp.float32(-jnp.inf)
        for b in range(_B):
            m = bat == b
            cnt = jnp.sum(m.astype(jnp.float32))
            s = jnp.sum(jnp.where(m, h, 0.0), axis=0)
            means.append(s / jnp.maximum(cnt, 1.0))
            maxs.append(jnp.max(jnp.where(m, h, neg), axis=0))
        x_mean = jnp.stack(means, axis=0)
        x_max = jnp.stack(maxs, axis=0)
        z = (jnp.dot(x_mean, wc1_ref[:_H, :], preferred_element_type=jnp.float32)
             + jnp.dot(x_max, wc1_ref[_H:, :], preferred_element_type=jnp.float32)
             + bc1_ref[...])
        z = jnp.maximum(z, 0.0)
        out_ref[...] = (jnp.dot(z, wc2_ref[...], preferred_element_type=jnp.float32)
                        + bc2_ref[...])

    return pl.pallas_call(
        body, out_shape=jax.ShapeDtypeStruct((_B, 2), jnp.float32))(
            agg, h1, batch2d, W2a, b2a, g2, be2, W2b, b2b, Wc1, bc1, Wc2, bc2)


def kernel(x, edge_index, batch, W1a, b1a, g1, be1, W1b, b1b,
           W2a, b2a, g2, be2, W2b, b2b, Wc1, bc1, Wc2, bc2):
    src, dst = edge_index[0], edge_index[1]
    pad = _EPAD - _E

    def split(idx, fill):
        flat = jnp.concatenate([idx, jnp.full((pad,), fill, jnp.int32)])
        n0 = _NS * _K0 * _CH
        c0 = flat[:n0].reshape(_NS, _K0, _CH)
        c1 = flat[n0:].reshape(_NS, _K1, _CH)
        tail0 = jnp.full((_NS, _KMAX - _K0, _CH), fill, jnp.int32)
        tail1 = jnp.full((_NS, _KMAX - _K1, _CH), fill, jnp.int32)
        return jnp.stack([jnp.concatenate([c0, tail0], axis=1),
                          jnp.concatenate([c1, tail1], axis=1)])

    src3 = split(src, 0)
    dst3 = split(dst, _N)
    zpadF = jnp.zeros((_NPAD, _F // 2), jnp.float32)
    zpadH = jnp.zeros((_NPAD, _H), jnp.float32)
    batch2d = batch.reshape(_N, 1)

    agg1L = _segment_sum_sc(x[:, :_F // 2], src3, dst3, zpadF)
    agg1L, xR = lax.optimization_barrier((agg1L, x[:, _F // 2:]))
    agg1R = _segment_sum_sc(xR, src3, dst3, zpadF)
    h1 = _mid_tc(agg1L, agg1R, x, W1a, b1a, g1, be1, W1b, b1b)
    agg2 = _segment_sum_sc(h1, src3, dst3, zpadH)
    return _head_tc(agg2, h1, batch2d, W2a, b2a, g2, be2, W2b, b2b,
                    Wc1, bc1, Wc2, bc2)


# speedup vs baseline: 1.2111x; 1.2111x over previous
import functools

import jax
import jax.numpy as jnp
from jax import lax
from jax.experimental import pallas as pl
from jax.experimental.pallas import tpu as pltpu
from jax.experimental.pallas import tpu_sc as plsc

_N = 10000
_F = 128
_H = 32
_B = 16
_E = 320000

_NC = 2
_NS = 16
_CH = 128
_NBUF = 4
_KTOT = -(-_E // (_NS * _CH * _NBUF)) * _NBUF
_EPAD = _NS * _KTOT * _CH
_K0 = 156
_K1 = _KTOT - _K0
_KMAX = max(_K0, _K1)
_NPAD = -(-(_N + 1) // (_NS * 8)) * _NS * 8
_RPW = _NPAD // _NS


def _segment_sum_sc(y, src3, dst3, zpad):
    width = y.shape[1]
    mesh = plsc.VectorSubcoreMesh(core_axis_name="c", subcore_axis_name="s")

    @functools.partial(
        pl.kernel,
        out_type=jax.ShapeDtypeStruct((_NC, _NPAD, width), jnp.float32),
        mesh=mesh,
        scratch_types=[
            pltpu.VMEM((_KMAX, _CH), jnp.int32),
            pltpu.VMEM((_KMAX, _CH), jnp.int32),
            pltpu.VMEM((_NBUF, _CH, width), jnp.float32),
            pltpu.VMEM_SHARED((_NPAD, width), jnp.float32),
        ] + [pltpu.SemaphoreType.DMA] * (2 * _NBUF),
        compiler_params=pltpu.CompilerParams(use_tc_tiling_on_sc=False),
    )
    def seg_kernel(y_hbm, src_hbm, dst_hbm, z_hbm, out_hbm,
                   src_v, dst_v, rows_v, acc_sh, *sems):
        gsem = sems[:_NBUF]
        ssem = sems[_NBUF:]
        cid = lax.axis_index("c")
        sid = lax.axis_index("s")
        r0 = sid * _RPW
        pltpu.sync_copy(z_hbm.at[pl.ds(r0, _RPW)], acc_sh.at[pl.ds(r0, _RPW)])
        pltpu.sync_copy(src_hbm.at[cid, sid], src_v)
        pltpu.sync_copy(dst_hbm.at[cid, sid], dst_v)
        plsc.subcore_barrier()

        n_iters = jnp.where(cid == 0, _K0 // _NBUF, _K1 // _NBUF)

        def body(i, carry):
            descs = []
            for b in range(_NBUF):
                j = i * _NBUF + b

                @pl.when(i >= 1)
                def _drain():
                    pltpu.make_async_copy(
                        rows_v.at[b], acc_sh.at[dst_v.at[0]], ssem[b]).wait()

                descs.append(pltpu.async_copy(
                    y_hbm.at[src_v.at[j]], rows_v.at[b], gsem[b]))
            for b in range(_NBUF):
                j = i * _NBUF + b
                descs[b].wait()
                pltpu.async_copy(rows_v.at[b], acc_sh.at[dst_v.at[j]],
                                 ssem[b], add=True)
            return carry

        lax.fori_loop(0, n_iters, body, 0)

        @pl.when(n_iters >= 1)
        def _final_drain():
            for b in range(_NBUF):
                pltpu.make_async_copy(
                    rows_v.at[b], acc_sh.at[dst_v.at[0]], ssem[b]).wait()

        plsc.subcore_barrier()
        pltpu.sync_copy(acc_sh.at[pl.ds(r0, _RPW)],
                        out_hbm.at[cid, pl.ds(r0, _RPW)])

    return seg_kernel(y, src3, dst3, zpad)


def _gin_layer(h, agg_ref, Wa, ba, g, be, Wb, bb):
    h = h + agg_ref[0, :_N, :] + agg_ref[1, :_N, :]
    h = jnp.dot(h, Wa, preferred_element_type=jnp.float32) + ba
    mu = jnp.mean(h, axis=0)
    var = jnp.mean((h - mu) ** 2, axis=0)
    h = (h - mu) / jnp.sqrt(var + 1e-5) * g + be
    h = jnp.maximum(h, 0.0)
    return jnp.dot(h, Wb, preferred_element_type=jnp.float32) + bb


def _mid_tc(aggL, aggR, x, W1a, b1a, g1, be1, W1b, b1b):
    def body(aggL_ref, aggR_ref, x_ref, w1a_ref, b1a_ref, g1_ref, be1_ref,
             w1b_ref, b1b_ref, h1_ref):
        agg = jnp.concatenate(
            [aggL_ref[0, :_N, :] + aggL_ref[1, :_N, :],
             aggR_ref[0, :_N, :] + aggR_ref[1, :_N, :]], axis=1)
        h = x_ref[...] + agg
        h = jnp.dot(h, w1a_ref[...], preferred_element_type=jnp.float32) + b1a_ref[...]
        mu = jnp.mean(h, axis=0)
        var = jnp.mean((h - mu) ** 2, axis=0)
        h = (h - mu) / jnp.sqrt(var + 1e-5) * g1_ref[...] + be1_ref[...]
        h = jnp.maximum(h, 0.0)
        h = jnp.dot(h, w1b_ref[...], preferred_element_type=jnp.float32) + b1b_ref[...]
        h1_ref[...] = jnp.maximum(h, 0.0)

    return pl.pallas_call(
        body, out_shape=jax.ShapeDtypeStruct((_N, _H), jnp.float32))(
            aggL, aggR, x, W1a, b1a, g1, be1, W1b, b1b)


def _head_tc(agg, h1, batch2d, W2a, b2a, g2, be2, W2b, b2b, Wc1, bc1, Wc2, bc2):
    def body(agg_ref, h1_ref, bat_ref, w2a_ref, b2a_ref, g2_ref, be2_ref,
             w2b_ref, b2b_ref, wc1_ref, bc1_ref, wc2_ref, bc2_ref, out_ref):
        h = _gin_layer(h1_ref[...], agg_ref, w2a_ref[...], b2a_ref[...],
                       g2_ref[...], be2_ref[...], w2b_ref[...], b2b_ref[...])
        bat = bat_ref[...]
        means = []
        maxs = []
        neg = jnp.float32(-jnp.inf)
        for b in range(_B):
            m = bat == b
            cnt = jnp.sum(m.astype(jnp.float32))
            s = jnp.sum(jnp.where(m, h, 0.0), axis=0)
            means.append(s / jnp.maximum(cnt, 1.0))
            maxs.append(jnp.max(jnp.where(m, h, neg), axis=0))
        x_mean = jnp.stack(means, axis=0)
        x_max = jnp.stack(maxs, axis=0)
        z = (jnp.dot(x_mean, wc1_ref[:_H, :], preferred_element_type=jnp.float32)
             + jnp.dot(x_max, wc1_ref[_H:, :], preferred_element_type=jnp.float32)
             + bc1_ref[...])
        z = jnp.maximum(z, 0.0)
        out_ref[...] = (jnp.dot(z, wc2_ref[...], preferred_element_type=jnp.float32)
                        + bc2_ref[...])

    return pl.pallas_call(
        body, out_shape=jax.ShapeDtypeStruct((_B, 2), jnp.float32))(
            agg, h1, batch2d, W2a, b2a, g2, be2, W2b, b2b, Wc1, bc1, Wc2, bc2)


def kernel(x, edge_index, batch, W1a, b1a, g1, be1, W1b, b1b,
           W2a, b2a, g2, be2, W2b, b2b, Wc1, bc1, Wc2, bc2):
    src, dst = edge_index[0], edge_index[1]
    pad = _EPAD - _E

    def split(idx, fill):
        flat = jnp.concatenate([idx, jnp.full((pad,), fill, jnp.int32)])
        n0 = _NS * _K0 * _CH
        c0 = flat[:n0].reshape(_NS, _K0, _CH)
        c1 = flat[n0:].reshape(_NS, _K1, _CH)
        tail0 = jnp.full((_NS, _KMAX - _K0, _CH), fill, jnp.int32)
        tail1 = jnp.full((_NS, _KMAX - _K1, _CH), fill, jnp.int32)
        return jnp.stack([jnp.concatenate([c0, tail0], axis=1),
                          jnp.concatenate([c1, tail1], axis=1)])

    src3 = split(src, 0)
    dst3 = split(dst, _N)
    zpadF = jnp.zeros((_NPAD, _F // 2), jnp.float32)
    zpadH = jnp.zeros((_NPAD, _H), jnp.float32)
    batch2d = batch.reshape(_N, 1)

    agg1L = _segment_sum_sc(x[:, :_F // 2], src3, dst3, zpadF)
    agg1L, xR = lax.optimization_barrier((agg1L, x[:, _F // 2:]))
    agg1R = _segment_sum_sc(xR, src3, dst3, zpadF)
    h1 = _mid_tc(agg1L, agg1R, x, W1a, b1a, g1, be1, W1b, b1b)
    agg2 = _segment_sum_sc(h1, src3, dst3, zpadH)
    return _head_tc(agg2, h1, batch2d, W2a, b2a, g2, be2, W2b, b2b,
                    Wc1, bc1, Wc2, bc2)


# speedup vs baseline: 1.2923x; 1.0670x over previous
import functools

import jax
import jax.numpy as jnp
from jax import lax
from jax.experimental import pallas as pl
from jax.experimental.pallas import tpu as pltpu
from jax.experimental.pallas import tpu_sc as plsc

_N = 10000
_F = 128
_H = 32
_B = 16
_E = 320000

_NC = 2
_NS = 16
_CH = 128
_NBUF = 4
_KTOT = -(-_E // (_NS * _CH * _NBUF)) * _NBUF
_EPAD = _NS * _KTOT * _CH
_K0 = 152
_K1 = _KTOT - _K0
_KMAX = max(_K0, _K1)
_NPAD = -(-(_N + 1) // (_NS * 8)) * _NS * 8
_RPW = _NPAD // _NS


def _segment_sum_sc(y, src3, dst3, zpad):
    width = y.shape[1]
    mesh = plsc.VectorSubcoreMesh(core_axis_name="c", subcore_axis_name="s")

    @functools.partial(
        pl.kernel,
        out_type=jax.ShapeDtypeStruct((_NC, _NPAD, width), jnp.float32),
        mesh=mesh,
        scratch_types=[
            pltpu.VMEM((_KMAX, _CH), jnp.int32),
            pltpu.VMEM((_KMAX, _CH), jnp.int32),
            pltpu.VMEM((_NBUF, _CH, width), jnp.float32),
            pltpu.VMEM_SHARED((_NPAD, width), jnp.float32),
        ] + [pltpu.SemaphoreType.DMA] * (2 * _NBUF),
        compiler_params=pltpu.CompilerParams(use_tc_tiling_on_sc=False),
    )
    def seg_kernel(y_hbm, src_hbm, dst_hbm, z_hbm, out_hbm,
                   src_v, dst_v, rows_v, acc_sh, *sems):
        gsem = sems[:_NBUF]
        ssem = sems[_NBUF:]
        cid = lax.axis_index("c")
        sid = lax.axis_index("s")
        r0 = sid * _RPW
        pltpu.sync_copy(z_hbm.at[pl.ds(r0, _RPW)], acc_sh.at[pl.ds(r0, _RPW)])
        pltpu.sync_copy(src_hbm.at[cid, sid], src_v)
        pltpu.sync_copy(dst_hbm.at[cid, sid], dst_v)
        plsc.subcore_barrier()

        n_iters = jnp.where(cid == 0, _K0 // _NBUF, _K1 // _NBUF)

        def body(i, carry):
            descs = []
            for b in range(_NBUF):
                j = i * _NBUF + b

                @pl.when(i >= 1)
                def _drain():
                    pltpu.make_async_copy(
                        rows_v.at[b], acc_sh.at[dst_v.at[0]], ssem[b]).wait()

                descs.append(pltpu.async_copy(
                    y_hbm.at[src_v.at[j]], rows_v.at[b], gsem[b]))
            for b in range(_NBUF):
                j = i * _NBUF + b
                descs[b].wait()
                pltpu.async_copy(rows_v.at[b], acc_sh.at[dst_v.at[j]],
                                 ssem[b], add=True)
            return carry

        lax.fori_loop(0, n_iters, body, 0)

        @pl.when(n_iters >= 1)
        def _final_drain():
            for b in range(_NBUF):
                pltpu.make_async_copy(
                    rows_v.at[b], acc_sh.at[dst_v.at[0]], ssem[b]).wait()

        plsc.subcore_barrier()
        pltpu.sync_copy(acc_sh.at[pl.ds(r0, _RPW)],
                        out_hbm.at[cid, pl.ds(r0, _RPW)])

    return seg_kernel(y, src3, dst3, zpad)


def _gin_layer(h, agg_ref, Wa, ba, g, be, Wb, bb):
    h = h + agg_ref[0, :_N, :] + agg_ref[1, :_N, :]
    h = jnp.dot(h, Wa, preferred_element_type=jnp.float32) + ba
    mu = jnp.mean(h, axis=0)
    var = jnp.mean((h - mu) ** 2, axis=0)
    h = (h - mu) / jnp.sqrt(var + 1e-5) * g + be
    h = jnp.maximum(h, 0.0)
    return jnp.dot(h, Wb, preferred_element_type=jnp.float32) + bb


def _mid_tc(aggL, aggR, x, W1a, b1a, g1, be1, W1b, b1b):
    def body(aggL_ref, aggR_ref, x_ref, w1a_ref, b1a_ref, g1_ref, be1_ref,
             w1b_ref, b1b_ref, h1_ref):
        agg = jnp.concatenate(
            [aggL_ref[0, :_N, :] + aggL_ref[1, :_N, :],
             aggR_ref[0, :_N, :] + aggR_ref[1, :_N, :]], axis=1)
        h = x_ref[...] + agg
        h = jnp.dot(h, w1a_ref[...], preferred_element_type=jnp.float32) + b1a_ref[...]
        mu = jnp.mean(h, axis=0)
        var = jnp.mean((h - mu) ** 2, axis=0)
        h = (h - mu) / jnp.sqrt(var + 1e-5) * g1_ref[...] + be1_ref[...]
        h = jnp.maximum(h, 0.0)
        h = jnp.dot(h, w1b_ref[...], preferred_element_type=jnp.float32) + b1b_ref[...]
        h1_ref[...] = jnp.maximum(h, 0.0)

    return pl.pallas_call(
        body, out_shape=jax.ShapeDtypeStruct((_N, _H), jnp.float32))(
            aggL, aggR, x, W1a, b1a, g1, be1, W1b, b1b)


def _head_tc(agg, h1, batch2d, W2a, b2a, g2, be2, W2b, b2b, Wc1, bc1, Wc2, bc2):
    def body(agg_ref, h1_ref, bat_ref, w2a_ref, b2a_ref, g2_ref, be2_ref,
             w2b_ref, b2b_ref, wc1_ref, bc1_ref, wc2_ref, bc2_ref, out_ref):
        h = _gin_layer(h1_ref[...], agg_ref, w2a_ref[...], b2a_ref[...],
                       g2_ref[...], be2_ref[...], w2b_ref[...], b2b_ref[...])
        bat = bat_ref[...]
        means = []
        maxs = []
        neg = jnp.float32(-jnp.inf)
        for b in range(_B):
            m = bat == b
            cnt = jnp.sum(m.astype(jnp.float32))
            s = jnp.sum(jnp.where(m, h, 0.0), axis=0)
            means.append(s / jnp.maximum(cnt, 1.0))
            maxs.append(jnp.max(jnp.where(m, h, neg), axis=0))
        x_mean = jnp.stack(means, axis=0)
        x_max = jnp.stack(maxs, axis=0)
        z = (jnp.dot(x_mean, wc1_ref[:_H, :], preferred_element_type=jnp.float32)
             + jnp.dot(x_max, wc1_ref[_H:, :], preferred_element_type=jnp.float32)
             + bc1_ref[...])
        z = jnp.maximum(z, 0.0)
        out_ref[...] = (jnp.dot(z, wc2_ref[...], preferred_element_type=jnp.float32)
                        + bc2_ref[...])

    return pl.pallas_call(
        body, out_shape=jax.ShapeDtypeStruct((_B, 2), jnp.float32))(
            agg, h1, batch2d, W2a, b2a, g2, be2, W2b, b2b, Wc1, bc1, Wc2, bc2)


def kernel(x, edge_index, batch, W1a, b1a, g1, be1, W1b, b1b,
           W2a, b2a, g2, be2, W2b, b2b, Wc1, bc1, Wc2, bc2):
    src, dst = edge_index[0], edge_index[1]
    pad = _EPAD - _E

    def split(idx, fill):
        flat = jnp.concatenate([idx, jnp.full((pad,), fill, jnp.int32)])
        n0 = _NS * _K0 * _CH
        c0 = flat[:n0].reshape(_NS, _K0, _CH)
        c1 = flat[n0:].reshape(_NS, _K1, _CH)
        tail0 = jnp.full((_NS, _KMAX - _K0, _CH), fill, jnp.int32)
        tail1 = jnp.full((_NS, _KMAX - _K1, _CH), fill, jnp.int32)
        return jnp.stack([jnp.concatenate([c0, tail0], axis=1),
                          jnp.concatenate([c1, tail1], axis=1)])

    src3 = split(src, 0)
    dst3 = split(dst, _N)
    zpadF = jnp.zeros((_NPAD, _F // 2), jnp.float32)
    zpadH = jnp.zeros((_NPAD, _H), jnp.float32)
    batch2d = batch.reshape(_N, 1)

    agg1L = _segment_sum_sc(x[:, :_F // 2], src3, dst3, zpadF)
    agg1L, xR = lax.optimization_barrier((agg1L, x[:, _F // 2:]))
    agg1R = _segment_sum_sc(xR, src3, dst3, zpadF)
    h1 = _mid_tc(agg1L, agg1R, x, W1a, b1a, g1, be1, W1b, b1b)
    agg2 = _segment_sum_sc(h1, src3, dst3, zpadH)
    return _head_tc(agg2, h1, batch2d, W2a, b2a, g2, be2, W2b, b2b,
                    Wc1, bc1, Wc2, bc2)
